# baseline (speedup 1.0000x reference)
import functools

import jax
import jax.numpy as jnp
from jax import lax
from jax.experimental import pallas as pl
from jax.experimental.pallas import tpu as pltpu
from jax.experimental.pallas import tpu_sc as plsc

N = 100000
D_IN = 128
P = 256
C = 20
MOMENTUM = 0.99

R = 10000
GRID = N // R

NW = 32
K = 256
NCHUNK = (N + K - 1) // K
SC_ITERS = (NCHUNK + NW - 1) // NW
ACCW = 160


def _tc_body(x_ref, w1_ref, b1_ref, w2_ref, b2_ref, scores_ref, feats_ref):
    feats = jnp.dot(x_ref[...], w1_ref[...],
                    preferred_element_type=jnp.float32) + b1_ref[...]
    feats_ref[...] = feats
    scores_ref[...] = jnp.dot(feats, w2_ref[...],
                              preferred_element_type=jnp.float32) + b2_ref[...]


def _sc_body(x_hbm, labels_hbm, out_hbm, xb, lb, acc, unitv):
    wid = lax.axis_index("s") * 2 + lax.axis_index("c")

    zero16 = jnp.zeros((16,), jnp.float32)
    unitv[pl.ds(0, 16)] = jnp.ones((16,), jnp.float32)
    for i in range(C):
        for j in range(ACCW // 16):
            acc[i, pl.ds(j * 16, 16)] = zero16

    for t in range(SC_ITERS):
        chunk = wid + t * NW

        @pl.when(chunk < NCHUNK)
        def _process():
            is_tail = chunk == NCHUNK - 1
            base = pl.multiple_of(jnp.where(is_tail, N - K, chunk * K), 32)
            start_g = jnp.where(is_tail, (NCHUNK * K - N) // 16, 0)
            pltpu.sync_copy(x_hbm.at[pl.ds(base, K)], xb)
            pltpu.sync_copy(labels_hbm.at[pl.ds(base, K)], lb)

            def group_body(g, carry):
                r0 = g * 16
                lvec = lb[pl.ds(r0, 16)]
                one = unitv[pl.ds(0, 16)]
                for i in range(16):
                    l = lvec[i]
                    for j in range(D_IN // 16):
                        v = xb[r0 + i, pl.ds(j * 16, 16)]
                        plsc.addupdate(acc.at[l, pl.ds(j * 16, 16)], v)
                    plsc.addupdate(acc.at[l, pl.ds(D_IN, 16)], one)
                return carry

            lax.fori_loop(start_g, K // 16, group_body, 0)

    pltpu.sync_copy(acc, out_hbm.at[wid])


_sc_segsum = functools.partial(
    pl.kernel,
    out_type=jax.ShapeDtypeStruct((NW, C, ACCW), jnp.float32),
    mesh=plsc.VectorSubcoreMesh(core_axis_name="c", subcore_axis_name="s"),
    scratch_types=[
        pltpu.VMEM((K, D_IN), jnp.float32),
        pltpu.VMEM((K,), jnp.int32),
        pltpu.VMEM((C, ACCW), jnp.float32),
        pltpu.VMEM((16,), jnp.float32),
    ],
)(_sc_body)


def _fin_body(m_ref, part_ref, w1_ref, b1_ref, proto_ref, out_ref):
    tot = part_ref[0]
    for w in range(1, NW):
        tot = tot + part_ref[w]
    sums_x = tot[:, :D_IN]
    cnt = tot[:, D_IN:D_IN + 1]
    means_x = sums_x / jnp.maximum(cnt, 1.0)
    means = jnp.dot(means_x, w1_ref[...],
                    preferred_element_type=jnp.float32) + b1_ref[...]
    m = m_ref[0, 0]
    proto = proto_ref[...]
    out_ref[...] = jnp.where(cnt > 0.0, proto * m + means * (1.0 - m), proto)


@jax.jit
def kernel(sparse_feats, labels, global_step, W1, b1, W2, b2, prototypes):
    gs = jnp.asarray(global_step).astype(jnp.float32)
    m = jnp.minimum(1.0 - 1.0 / (1.0 + gs), MOMENTUM).reshape(1, 1)

    partials = _sc_segsum(sparse_feats, labels)

    scores, feats = pl.pallas_call(
        _tc_body,
        grid=(GRID,),
        in_specs=[
            pl.BlockSpec((R, D_IN), lambda i: (i, 0)),
            pl.BlockSpec((D_IN, P), lambda i: (0, 0)),
            pl.BlockSpec((1, P), lambda i: (0, 0)),
            pl.BlockSpec((P, C), lambda i: (0, 0)),
            pl.BlockSpec((1, C), lambda i: (0, 0)),
        ],
        out_specs=[
            pl.BlockSpec((R, C), lambda i: (i, 0)),
            pl.BlockSpec((R, P), lambda i: (i, 0)),
        ],
        out_shape=[
            jax.ShapeDtypeStruct((N, C), jnp.float32),
            jax.ShapeDtypeStruct((N, P), jnp.float32),
        ],
    )(sparse_feats, W1, b1.reshape(1, P), W2, b2.reshape(1, C))

    newproto = pl.pallas_call(
        _fin_body,
        in_specs=[
            pl.BlockSpec(memory_space=pltpu.SMEM),
            pl.BlockSpec(memory_space=pltpu.VMEM),
            pl.BlockSpec(memory_space=pltpu.VMEM),
            pl.BlockSpec(memory_space=pltpu.VMEM),
            pl.BlockSpec(memory_space=pltpu.VMEM),
        ],
        out_specs=pl.BlockSpec(memory_space=pltpu.VMEM),
        out_shape=jax.ShapeDtypeStruct((C, P), jnp.float32),
    )(m, partials, W1, b1.reshape(1, P), prototypes)

    return scores, feats, prototypes, newproto


# speedup vs baseline: 1.2671x; 1.2671x over previous
import functools

import jax
import jax.numpy as jnp
from jax import lax
from jax.experimental import pallas as pl
from jax.experimental.pallas import tpu as pltpu
from jax.experimental.pallas import tpu_sc as plsc

N = 100000
D_IN = 128
P = 256
C = 20
MOMENTUM = 0.99

R = 10000
GRID = N // R

NW = 32
K = 256
NCHUNK = (N + K - 1) // K
SC_ITERS = (NCHUNK + NW - 1) // NW
CPAD = 24


def _tc_body(x_ref, w1_ref, b1_ref, w2_ref, b2_ref, scores_ref, feats_ref):
    feats = jnp.dot(x_ref[...], w1_ref[...],
                    preferred_element_type=jnp.float32) + b1_ref[...]
    feats_ref[...] = feats
    scores_ref[...] = jnp.dot(feats, w2_ref[...],
                              preferred_element_type=jnp.float32) + b2_ref[...]


def _sc_body(x_hbm, labels_hbm, out_hbm, outc_hbm, xb, lb2, zb, cnt2, unitv,
             shacc):
    core = lax.axis_index("c")
    sub = lax.axis_index("s")
    wid = sub * 2 + core

    zero16 = jnp.zeros((16,), jnp.float32)
    for i in range(CPAD):
        for j in range(D_IN // 16):
            zb[i, pl.ds(j * 16, 16)] = zero16
        cnt2[i, pl.ds(0, 16)] = zero16
    unitv[pl.ds(0, 16)] = jnp.ones((16,), jnp.float32)

    @pl.when(sub == 0)
    def _zero_shared():
        pltpu.sync_copy(zb, shacc)

    plsc.subcore_barrier()

    for t in range(SC_ITERS):
        chunk = wid + t * NW

        @pl.when(chunk < NCHUNK)
        def _process():
            is_tail = chunk == NCHUNK - 1
            base = pl.multiple_of(jnp.where(is_tail, N - K, chunk * K), 32)
            pltpu.sync_copy(labels_hbm.at[pl.ds(base, 128)], lb2.at[0])
            pltpu.sync_copy(labels_hbm.at[pl.ds(base + 128, 128)], lb2.at[1])

            @pl.when(is_tail)
            def _mask_tail():
                dummy = jnp.full((16,), CPAD - 1, jnp.int32)
                for g in range(6):
                    lb2[0, pl.ds(g * 16, 16)] = dummy

            pltpu.sync_copy(x_hbm.at[pl.ds(base, K)], xb)
            for s in range(2):
                pltpu.sync_copy(xb.at[pl.ds(s * 128, 128)],
                                shacc.at[lb2.at[s]], add=True)

            one = unitv[pl.ds(0, 16)]
            for s in range(2):
                for g in range(8):
                    lvec = lb2[s, pl.ds(g * 16, 16)]
                    for i in range(16):
                        l = lvec[i]
                        plsc.addupdate(cnt2.at[l, pl.ds(0, 16)], one)

    plsc.subcore_barrier()

    @pl.when(sub == 0)
    def _write_shared():
        pltpu.sync_copy(shacc, out_hbm.at[core])

    pltpu.sync_copy(cnt2, outc_hbm.at[wid])


_sc_segsum = functools.partial(
    pl.kernel,
    out_type=[jax.ShapeDtypeStruct((2, CPAD, D_IN), jnp.float32),
              jax.ShapeDtypeStruct((NW, CPAD, 16), jnp.float32)],
    mesh=plsc.VectorSubcoreMesh(core_axis_name="c", subcore_axis_name="s"),
    scratch_types=[
        pltpu.VMEM((K, D_IN), jnp.float32),
        pltpu.VMEM((2, 128), jnp.int32),
        pltpu.VMEM((CPAD, D_IN), jnp.float32),
        pltpu.VMEM((CPAD, 16), jnp.float32),
        pltpu.VMEM((16,), jnp.float32),
        pltpu.VMEM_SHARED((CPAD, D_IN), jnp.float32),
    ],
)(_sc_body)


def _fin_body(m_ref, part_ref, partc_ref, w1_ref, b1_ref, proto_ref, out_ref):
    tot = part_ref[0] + part_ref[1]
    totc = partc_ref[0]
    for w in range(1, NW):
        totc = totc + partc_ref[w]
    sums_x = tot[:C, :]
    cnt = totc[:C, 0:1]
    means_x = sums_x / jnp.maximum(cnt, 1.0)
    means = jnp.dot(means_x, w1_ref[...],
                    preferred_element_type=jnp.float32) + b1_ref[...]
    m = m_ref[0, 0]
    proto = proto_ref[...]
    out_ref[...] = jnp.where(cnt > 0.0, proto * m + means * (1.0 - m), proto)


@jax.jit
def kernel(sparse_feats, labels, global_step, W1, b1, W2, b2, prototypes):
    gs = jnp.asarray(global_step).astype(jnp.float32)
    m = jnp.minimum(1.0 - 1.0 / (1.0 + gs), MOMENTUM).reshape(1, 1)

    partials, partial_counts = _sc_segsum(sparse_feats, labels)

    scores, feats = pl.pallas_call(
        _tc_body,
        grid=(GRID,),
        in_specs=[
            pl.BlockSpec((R, D_IN), lambda i: (i, 0)),
            pl.BlockSpec((D_IN, P), lambda i: (0, 0)),
            pl.BlockSpec((1, P), lambda i: (0, 0)),
            pl.BlockSpec((P, C), lambda i: (0, 0)),
            pl.BlockSpec((1, C), lambda i: (0, 0)),
        ],
        out_specs=[
            pl.BlockSpec((R, C), lambda i: (i, 0)),
            pl.BlockSpec((R, P), lambda i: (i, 0)),
        ],
        out_shape=[
            jax.ShapeDtypeStruct((N, C), jnp.float32),
            jax.ShapeDtypeStruct((N, P), jnp.float32),
        ],
    )(sparse_feats, W1, b1.reshape(1, P), W2, b2.reshape(1, C))

    newproto = pl.pallas_call(
        _fin_body,
        in_specs=[
            pl.BlockSpec(memory_space=pltpu.SMEM),
            pl.BlockSpec(memory_space=pltpu.VMEM),
            pl.BlockSpec(memory_space=pltpu.VMEM),
            pl.BlockSpec(memory_space=pltpu.VMEM),
            pl.BlockSpec(memory_space=pltpu.VMEM),
            pl.BlockSpec(memory_space=pltpu.VMEM),
        ],
        out_specs=pl.BlockSpec(memory_space=pltpu.VMEM),
        out_shape=jax.ShapeDtypeStruct((C, P), jnp.float32),
    )(m, partials, partial_counts, W1, b1.reshape(1, P), prototypes)

    return scores, feats, prototypes, newproto


# speedup vs baseline: 1.2884x; 1.0168x over previous
import functools

import jax
import jax.numpy as jnp
from jax import lax
from jax.experimental import pallas as pl
from jax.experimental.pallas import tpu as pltpu
from jax.experimental.pallas import tpu_sc as plsc

N = 100000
D_IN = 128
P = 256
C = 20
MOMENTUM = 0.99

R = 10000
GRID = N // R

NW = 32
K = 256
NCHUNK = (N + K - 1) // K
SC_ITERS = (NCHUNK + NW - 1) // NW
CPAD = 24
K2 = 256
TAIL_SKIP = NCHUNK * K - N
LAST_ROUND_WIDS = NCHUNK - (SC_ITERS - 1) * NW


def _tc_body(x_ref, w1_ref, b1_ref, w2_ref, b2_ref, scores_ref, feats_ref):
    feats = jnp.dot(x_ref[...], w1_ref[...],
                    preferred_element_type=jnp.float32) + b1_ref[...]
    feats_ref[...] = feats
    scores_ref[...] = jnp.dot(feats, w2_ref[...],
                              preferred_element_type=jnp.float32) + b2_ref[...]


def _sc_body(x_hbm, labels_hbm, out_hbm, outc_hbm, xb, lb2, zb, cnt2, unitv,
             shacc, semx0, semx1, seml0, seml1, semsc):
    core = lax.axis_index("c")
    sub = lax.axis_index("s")
    wid = sub * 2 + core
    semx = [semx0, semx1]
    seml = [seml0, seml1]

    zero16 = jnp.zeros((16,), jnp.float32)
    for i in range(CPAD):
        for j in range(D_IN // 16):
            zb[i, pl.ds(j * 16, 16)] = zero16
        cnt2[i, pl.ds(0, 16)] = zero16
    unitv[pl.ds(0, 16)] = jnp.ones((16,), jnp.float32)

    @pl.when(sub == 0)
    def _zero_shared():
        pltpu.sync_copy(zb, shacc)

    plsc.subcore_barrier()

    def chunk_base(t):
        chunk = wid + t * NW
        is_tail = chunk == NCHUNK - 1
        return is_tail, pl.multiple_of(jnp.where(is_tail, N - K, chunk * K), 32)

    def prefetch(t, b):
        _, base = chunk_base(t)
        pltpu.async_copy(x_hbm.at[pl.ds(base, K)], xb.at[b], semx[b])
        for s2 in range(2):
            pltpu.async_copy(labels_hbm.at[pl.ds(base + s2 * 128, 128)],
                             lb2.at[b].at[s2], seml[b])

    def process(t, b):
        is_tail, base = chunk_base(t)
        pltpu.make_async_copy(x_hbm.at[pl.ds(base, K)], xb.at[b], semx[b]).wait()
        for s2 in range(2):
            pltpu.make_async_copy(labels_hbm.at[pl.ds(base + s2 * 128, 128)],
                                  lb2.at[b].at[s2], seml[b]).wait()

        @pl.when(is_tail)
        def _mask_tail():
            dummy = jnp.full((16,), CPAD - 1, jnp.int32)
            for g in range(TAIL_SKIP // 16):
                lb2[b, 0, pl.ds(g * 16, 16)] = dummy

        for s in range(2):
            pltpu.async_copy(xb.at[b].at[pl.ds(s * 128, 128)],
                             shacc.at[lb2.at[b].at[s]], semsc, add=True)

        one = unitv[pl.ds(0, 16)]
        for s in range(2):
            for g in range(8):
                lvec = lb2[b, s, pl.ds(g * 16, 16)]
                for i in range(16):
                    l = lvec[i]
                    plsc.addupdate(cnt2.at[l, pl.ds(0, 16)], one)

        for s in range(2):
            pltpu.make_async_copy(xb.at[b].at[pl.ds(s * 128, 128)],
                                  shacc.at[lb2.at[b].at[s]], semsc).wait()

    prefetch(0, 0)
    for t in range(SC_ITERS - 1):
        nxt = t + 1
        if nxt < SC_ITERS - 1:
            prefetch(nxt, nxt % 2)
        else:
            @pl.when(wid < LAST_ROUND_WIDS)
            def _pf_last():
                prefetch(nxt, nxt % 2)
        process(t, t % 2)

    @pl.when(wid < LAST_ROUND_WIDS)
    def _last():
        process(SC_ITERS - 1, (SC_ITERS - 1) % 2)

    plsc.subcore_barrier()

    @pl.when(sub == 0)
    def _write_shared():
        pltpu.sync_copy(shacc, out_hbm.at[core])

    pltpu.sync_copy(cnt2, outc_hbm.at[wid])


_sc_segsum = functools.partial(
    pl.kernel,
    out_type=[jax.ShapeDtypeStruct((2, CPAD, D_IN), jnp.float32),
              jax.ShapeDtypeStruct((NW, CPAD, 16), jnp.float32)],
    mesh=plsc.VectorSubcoreMesh(core_axis_name="c", subcore_axis_name="s"),
    scratch_types=[
        pltpu.VMEM((2, K, D_IN), jnp.float32),
        pltpu.VMEM((2, 2, 128), jnp.int32),
        pltpu.VMEM((CPAD, D_IN), jnp.float32),
        pltpu.VMEM((CPAD, 16), jnp.float32),
        pltpu.VMEM((16,), jnp.float32),
        pltpu.VMEM_SHARED((CPAD, D_IN), jnp.float32),
        pltpu.SemaphoreType.DMA,
        pltpu.SemaphoreType.DMA,
        pltpu.SemaphoreType.DMA,
        pltpu.SemaphoreType.DMA,
        pltpu.SemaphoreType.DMA,
    ],
)(_sc_body)


def _fin_body(m_ref, part_ref, partc_ref, w1_ref, b1_ref, proto_ref, out_ref):
    tot = part_ref[0] + part_ref[1]
    totc = partc_ref[0]
    for w in range(1, NW):
        totc = totc + partc_ref[w]
    sums_x = tot[:C, :]
    cnt = totc[:C, 0:1]
    means_x = sums_x / jnp.maximum(cnt, 1.0)
    means = jnp.dot(means_x, w1_ref[...],
                    preferred_element_type=jnp.float32) + b1_ref[...]
    m = m_ref[0, 0]
    proto = proto_ref[...]
    out_ref[...] = jnp.where(cnt > 0.0, proto * m + means * (1.0 - m), proto)


@jax.jit
def kernel(sparse_feats, labels, global_step, W1, b1, W2, b2, prototypes):
    gs = jnp.asarray(global_step).astype(jnp.float32)
    m = jnp.minimum(1.0 - 1.0 / (1.0 + gs), MOMENTUM).reshape(1, 1)

    partials, partial_counts = _sc_segsum(sparse_feats, labels)

    scores, feats = pl.pallas_call(
        _tc_body,
        grid=(GRID,),
        in_specs=[
            pl.BlockSpec((R, D_IN), lambda i: (i, 0)),
            pl.BlockSpec((D_IN, P), lambda i: (0, 0)),
            pl.BlockSpec((1, P), lambda i: (0, 0)),
            pl.BlockSpec((P, C), lambda i: (0, 0)),
            pl.BlockSpec((1, C), lambda i: (0, 0)),
        ],
        out_specs=[
            pl.BlockSpec((R, C), lambda i: (i, 0)),
            pl.BlockSpec((R, P), lambda i: (i, 0)),
        ],
        out_shape=[
            jax.ShapeDtypeStruct((N, C), jnp.float32),
            jax.ShapeDtypeStruct((N, P), jnp.float32),
        ],
    )(sparse_feats, W1, b1.reshape(1, P), W2, b2.reshape(1, C))

    newproto = pl.pallas_call(
        _fin_body,
        in_specs=[
            pl.BlockSpec(memory_space=pltpu.SMEM),
            pl.BlockSpec(memory_space=pltpu.VMEM),
            pl.BlockSpec(memory_space=pltpu.VMEM),
            pl.BlockSpec(memory_space=pltpu.VMEM),
            pl.BlockSpec(memory_space=pltpu.VMEM),
            pl.BlockSpec(memory_space=pltpu.VMEM),
        ],
        out_specs=pl.BlockSpec(memory_space=pltpu.VMEM),
        out_shape=jax.ShapeDtypeStruct((C, P), jnp.float32),
    )(m, partials, partial_counts, W1, b1.reshape(1, P), prototypes)

    return scores, feats, prototypes, newproto
